# baseline (speedup 1.0000x reference)
import functools

import jax
import jax.numpy as jnp
from jax import lax
from jax.experimental import pallas as pl
from jax.experimental.pallas import tpu as pltpu
from jax.experimental.pallas import tpu_sc as plsc

DIM = 64
NC = 2
NS = 16
NW = NC * NS
CHUNK = 128


def _make_gather(n_idx):
    assert n_idx % (NW * CHUNK) == 0
    per_w = n_idx // NW
    n_chunks = per_w // CHUNK
    mesh = plsc.VectorSubcoreMesh(core_axis_name="c", subcore_axis_name="s")

    @functools.partial(
        pl.kernel,
        out_type=jax.ShapeDtypeStruct((n_idx, DIM), jnp.float32),
        mesh=mesh,
        scratch_types=[
            pltpu.VMEM((CHUNK,), jnp.int32),
            pltpu.VMEM((CHUNK, DIM), jnp.float32),
            pltpu.SemaphoreType.DMA,
        ],
        compiler_params=pltpu.CompilerParams(use_tc_tiling_on_sc=False),
    )
    def gather_kernel(idx_hbm, table_hbm, out_hbm, idx_v, rows_v, sem):
        wid = lax.axis_index("s") * NC + lax.axis_index("c")
        base = wid * per_w

        @pl.loop(0, n_chunks)
        def _(j):
            off = base + j * CHUNK
            pltpu.sync_copy(idx_hbm.at[pl.ds(off, CHUNK)], idx_v)
            pltpu.async_copy(table_hbm.at[idx_v], rows_v, sem).wait()
            pltpu.sync_copy(rows_v, out_hbm.at[pl.ds(off, CHUNK)])

    return gather_kernel


@jax.jit
def kernel(x, action_emb_weight):
    b, h = x.shape
    flat_idx = x.reshape(-1).astype(jnp.int32)
    out = _make_gather(b * h)(flat_idx, action_emb_weight)
    return out.reshape(b, h, DIM)


# speedup vs baseline: 1.1930x; 1.1930x over previous
import functools

import jax
import jax.numpy as jnp
from jax import lax
from jax.experimental import pallas as pl
from jax.experimental.pallas import tpu as pltpu
from jax.experimental.pallas import tpu_sc as plsc

DIM = 64
NC = 2
NS = 16
NW = NC * NS
CHUNK = 128
NBUF = 8


def _make_gather(n_idx):
    assert n_idx % (NW * CHUNK) == 0
    per_w = n_idx // NW
    n_chunks = per_w // CHUNK
    assert n_chunks % NBUF == 0
    n_groups = n_chunks // NBUF
    mesh = plsc.VectorSubcoreMesh(core_axis_name="c", subcore_axis_name="s")

    scratch = (
        [pltpu.VMEM((n_chunks, CHUNK), jnp.int32)]
        + [pltpu.VMEM((CHUNK, DIM), jnp.float32) for _ in range(NBUF)]
        + [pltpu.SemaphoreType.DMA for _ in range(2 * NBUF)]
    )

    @functools.partial(
        pl.kernel,
        out_type=jax.ShapeDtypeStruct((n_idx, DIM), jnp.float32),
        mesh=mesh,
        scratch_types=scratch,
        compiler_params=pltpu.CompilerParams(use_tc_tiling_on_sc=False),
    )
    def gather_kernel(idx_hbm, table_hbm, out_hbm, idx_v, *bufs_sems):
        rows = bufs_sems[:NBUF]
        gsem = bufs_sems[NBUF:2 * NBUF]
        ssem = bufs_sems[2 * NBUF:]
        wid = lax.axis_index("s") * NC + lax.axis_index("c")
        base = wid * per_w

        pltpu.sync_copy(idx_hbm.at[pl.ds(wid * n_chunks, n_chunks)], idx_v)

        def start_gather(j, b):
            pltpu.make_async_copy(
                table_hbm.at[idx_v.at[j]], rows[b], gsem[b]).start()

        def wait_gather(j, b):
            pltpu.make_async_copy(
                table_hbm.at[idx_v.at[j]], rows[b], gsem[b]).wait()

        def start_store(j, b):
            pltpu.make_async_copy(
                rows[b], out_hbm.at[pl.ds(base + j * CHUNK, CHUNK)],
                ssem[b]).start()

        def wait_store(j, b):
            pltpu.make_async_copy(
                rows[b], out_hbm.at[pl.ds(base + j * CHUNK, CHUNK)],
                ssem[b]).wait()

        for b in range(NBUF):
            start_gather(b, b)

        @pl.loop(0, n_groups - 1)
        def _(g):
            j0 = g * NBUF
            for b in range(NBUF):
                wait_gather(j0 + b, b)
                start_store(j0 + b, b)
            for b in range(NBUF):
                wait_store(j0 + b, b)
                start_gather(j0 + NBUF + b, b)

        j0 = (n_groups - 1) * NBUF
        for b in range(NBUF):
            wait_gather(j0 + b, b)
            start_store(j0 + b, b)
        for b in range(NBUF):
            wait_store(j0 + b, b)

    return gather_kernel


@jax.jit
def kernel(x, action_emb_weight):
    b, h = x.shape
    flat_idx = x.reshape(-1, CHUNK).astype(jnp.int32)
    out = _make_gather(b * h)(flat_idx, action_emb_weight)
    return out.reshape(b, h, DIM)
